# Initial kernel scaffold; baseline (speedup 1.0000x reference)
import jax
import jax.numpy as jnp
from jax.experimental import pallas as pl


def kernel(variable_emb, edge_emb, constraint_emb, W_left, b_left, W_edge, W_right, W_join, b_join, W_merge, b_merge, e_u, e_v):
    raise NotImplementedError("write your pallas kernel here")


# speedup vs baseline: 2.7588x; 2.7588x over previous
import functools

import jax
import jax.numpy as jnp
from jax import lax
from jax.experimental import pallas as pl
from jax.experimental.pallas import tpu as pltpu
from jax.experimental.pallas import tpu_sc as plsc

NC = 2
NS = 16
NW = NC * NS


def _ln(x, eps=1e-5):
    m = jnp.mean(x, axis=-1, keepdims=True)
    v = jnp.mean((x - m) ** 2, axis=-1, keepdims=True)
    return (x - m) * lax.rsqrt(v + eps)


def _dotT(x, w):
    return lax.dot_general(x, w, (((1,), (1,)), ((), ())),
                           preferred_element_type=jnp.float32)


def _transform_body(x_ref, w_ref, b_ref, o_ref):
    o_ref[...] = _dotT(_ln(x_ref[...]), w_ref[...]) + b_ref[...]


def _transform(x, w, b, blk):
    n, d = x.shape
    return pl.pallas_call(
        _transform_body,
        grid=(n // blk,),
        in_specs=[pl.BlockSpec((blk, d), lambda i: (i, 0)),
                  pl.BlockSpec((d, d), lambda i: (0, 0)),
                  pl.BlockSpec((1, d), lambda i: (0, 0))],
        out_specs=pl.BlockSpec((blk, d), lambda i: (i, 0)),
        out_shape=jax.ShapeDtypeStruct((n, d), jnp.float32),
    )(x, w, b.reshape(1, d))


def _joint_body(a_ref, e_ref, c_ref, w_ref, b_ref, o_ref):
    g = _ln(jnp.maximum(a_ref[...] + e_ref[...] + c_ref[...], 0.0))
    o_ref[...] = _ln(_dotT(g, w_ref[...]) + b_ref[...])


def _joint(va, ea, ca, w, b, blk):
    n, d = va.shape
    return pl.pallas_call(
        _joint_body,
        grid=(n // blk,),
        in_specs=[pl.BlockSpec((blk, d), lambda i: (i, 0)),
                  pl.BlockSpec((blk, d), lambda i: (i, 0)),
                  pl.BlockSpec((blk, d), lambda i: (i, 0)),
                  pl.BlockSpec((d, d), lambda i: (0, 0)),
                  pl.BlockSpec((1, d), lambda i: (0, 0))],
        out_specs=pl.BlockSpec((blk, d), lambda i: (i, 0)),
        out_shape=jax.ShapeDtypeStruct((n, d), jnp.float32),
    )(va, ea, ca, w, b.reshape(1, d))


def _merge_body(base_ref, agg_ref, w_ref, b_ref, o_ref):
    d = base_ref.shape[1]
    h = (_dotT(base_ref[...], w_ref[:, :d]) +
         _dotT(agg_ref[...], w_ref[:, d:]) + b_ref[...])
    o_ref[...] = base_ref[...] + _ln(jnp.maximum(h, 0.0))


def _merge(base, agg, w, b, blk):
    n, d = base.shape
    return pl.pallas_call(
        _merge_body,
        grid=(n // blk,),
        in_specs=[pl.BlockSpec((blk, d), lambda i: (i, 0)),
                  pl.BlockSpec((blk, d), lambda i: (i, 0)),
                  pl.BlockSpec((d, 2 * d), lambda i: (0, 0)),
                  pl.BlockSpec((1, d), lambda i: (0, 0))],
        out_specs=pl.BlockSpec((blk, d), lambda i: (i, 0)),
        out_shape=jax.ShapeDtypeStruct((n, d), jnp.float32),
    )(base, agg, w, b.reshape(1, d))


def _sc_gather(table, idx):
    n, d = table.shape
    ne = idx.shape[0]
    chunk = 128
    nchunks = ne // chunk
    per_w = nchunks // NW
    extra = nchunks - per_w * NW
    mesh = plsc.VectorSubcoreMesh(core_axis_name="c", subcore_axis_name="s")

    @functools.partial(
        pl.kernel, mesh=mesh,
        out_type=jax.ShapeDtypeStruct((ne, d), jnp.float32),
        scratch_types=[pltpu.VMEM((chunk,), jnp.int32),
                       pltpu.VMEM((chunk, d), jnp.float32),
                       pltpu.SemaphoreType.DMA],
    )
    def k(table_hbm, idx_hbm, out_hbm, idx_v, buf_v, sem):
        wid = lax.axis_index("s") * NC + lax.axis_index("c")

        def do_chunk(j):
            base = j * chunk
            pltpu.sync_copy(idx_hbm.at[pl.ds(base, chunk)], idx_v)
            pltpu.async_copy(table_hbm.at[idx_v], buf_v, sem).wait()
            pltpu.sync_copy(buf_v, out_hbm.at[pl.ds(base, chunk), :])

        def body(i, carry):
            do_chunk(wid * per_w + i)
            return carry
        lax.fori_loop(0, per_w, body, 0)

        @pl.when(wid < extra)
        def _():
            do_chunk(NW * per_w + wid)

    return k(table, idx)


def _sc_segsum(joint, idx, nseg):
    ne, d = joint.shape
    dh = d // NC
    chunk = 128
    nchunks = ne // chunk
    per_s = nchunks // NS
    extra = nchunks - per_s * NS
    rows_per_s = (nseg // NS) // 8 * 8
    tail = nseg - rows_per_s * NS
    zr = 16
    mesh = plsc.VectorSubcoreMesh(core_axis_name="c", subcore_axis_name="s")

    @functools.partial(
        pl.kernel, mesh=mesh,
        out_type=jax.ShapeDtypeStruct((nseg, d), jnp.float32),
        scratch_types=[pltpu.VMEM((chunk,), jnp.int32),
                       pltpu.VMEM((chunk, dh), jnp.float32),
                       pltpu.VMEM((zr, dh), jnp.float32),
                       pltpu.VMEM_SHARED((nseg, dh), jnp.float32),
                       pltpu.SemaphoreType.DMA],
    )
    def k(joint_hbm, idx_hbm, out_hbm, idx_v, buf_v, zbuf, acc_sh, sem):
        c = lax.axis_index("c")
        s = lax.axis_index("s")
        row_base = s * rows_per_s
        nz = (rows_per_s + jnp.where(s == NS - 1, tail, 0)) // zr

        def zrow(r, carry):
            def zcol(q, carry2):
                zbuf[r, pl.ds(q * 16, 16)] = jnp.zeros((16,), jnp.float32)
                return carry2
            return lax.fori_loop(0, dh // 16, zcol, carry)
        lax.fori_loop(0, zr, zrow, 0)

        def zdma(t, carry):
            pltpu.sync_copy(zbuf, acc_sh.at[pl.ds(row_base + t * zr, zr)])
            return carry
        lax.fori_loop(0, nz, zdma, 0)
        plsc.subcore_barrier()

        def do_chunk(j):
            base = j * chunk
            pltpu.sync_copy(idx_hbm.at[pl.ds(base, chunk)], idx_v)
            pltpu.sync_copy(joint_hbm.at[pl.ds(base, chunk), pl.ds(c * dh, dh)],
                            buf_v)
            pltpu.sync_copy(buf_v, acc_sh.at[idx_v], add=True)

        def body(i, carry):
            do_chunk(i * NS + s)
            return carry
        lax.fori_loop(0, per_s, body, 0)

        @pl.when(s < extra)
        def _():
            do_chunk(NS * per_s + s)
        plsc.subcore_barrier()

        pltpu.sync_copy(
            acc_sh.at[pl.ds(row_base, rows_per_s)],
            out_hbm.at[pl.ds(row_base, rows_per_s), pl.ds(c * dh, dh)])

        if tail:
            @pl.when(s == NS - 1)
            def _():
                tb = NS * rows_per_s
                pltpu.sync_copy(
                    acc_sh.at[pl.ds(tb, tail)],
                    out_hbm.at[pl.ds(tb, tail), pl.ds(c * dh, dh)])

    return k(joint, idx)


def kernel(variable_emb, edge_emb, constraint_emb, W_left, b_left, W_edge,
           W_right, W_join, b_join, W_merge, b_merge, e_u, e_v):
    nu, d = variable_emb.shape
    nv = constraint_emb.shape[0]
    zb = jnp.zeros((d,), jnp.float32)

    var_t = _transform(variable_emb, W_left, b_left, 2000)
    edge_t = _transform(edge_emb, W_edge, zb, 1000)
    con_t = _transform(constraint_emb, W_right, zb, 2000)

    var_agg = _sc_gather(var_t, e_u)

    con_agg = _sc_gather(con_t, e_v)
    joint1 = _joint(var_agg, edge_t, con_agg, W_join, b_join, 1000)
    agg1 = _sc_segsum(joint1, e_v, nv)
    con2 = _merge(con_t, agg1, W_merge, b_merge, 2000)

    con_t2 = _transform(con2, W_right, zb, 2000)
    con_agg2 = _sc_gather(con_t2, e_v)
    joint2 = _joint(var_agg, edge_t, con_agg2, W_join, b_join, 1000)
    agg2 = _sc_segsum(joint2, e_u, nu)
    var2 = _merge(var_t, agg2, W_merge, b_merge, 2000)

    return (var2, con2)


# speedup vs baseline: 3.3571x; 1.2169x over previous
import functools

import jax
import jax.numpy as jnp
from jax import lax
from jax.experimental import pallas as pl
from jax.experimental.pallas import tpu as pltpu
from jax.experimental.pallas import tpu_sc as plsc

NC = 2
NS = 16
NW = NC * NS


def _ln(x, eps=1e-5):
    m = jnp.mean(x, axis=-1, keepdims=True)
    v = jnp.mean((x - m) ** 2, axis=-1, keepdims=True)
    return (x - m) * lax.rsqrt(v + eps)


def _dotT(x, w):
    return lax.dot_general(x, w, (((1,), (1,)), ((), ())),
                           preferred_element_type=jnp.float32)


def _transform_body(x_ref, w_ref, b_ref, o_ref):
    o_ref[...] = _dotT(_ln(x_ref[...]), w_ref[...]) + b_ref[...]


def _transform(x, w, b, blk):
    n, d = x.shape
    return pl.pallas_call(
        _transform_body,
        grid=(n // blk,),
        in_specs=[pl.BlockSpec((blk, d), lambda i: (i, 0)),
                  pl.BlockSpec((d, d), lambda i: (0, 0)),
                  pl.BlockSpec((1, d), lambda i: (0, 0))],
        out_specs=pl.BlockSpec((blk, d), lambda i: (i, 0)),
        out_shape=jax.ShapeDtypeStruct((n, d), jnp.float32),
    )(x, w, b.reshape(1, d))


def _joint_body(a_ref, e_ref, c_ref, w_ref, b_ref, o_ref):
    g = _ln(jnp.maximum(a_ref[...] + e_ref[...] + c_ref[...], 0.0))
    o_ref[...] = _ln(_dotT(g, w_ref[...]) + b_ref[...])


def _joint(va, ea, ca, w, b, blk):
    n, d = va.shape
    return pl.pallas_call(
        _joint_body,
        grid=(n // blk,),
        in_specs=[pl.BlockSpec((blk, d), lambda i: (i, 0)),
                  pl.BlockSpec((blk, d), lambda i: (i, 0)),
                  pl.BlockSpec((blk, d), lambda i: (i, 0)),
                  pl.BlockSpec((d, d), lambda i: (0, 0)),
                  pl.BlockSpec((1, d), lambda i: (0, 0))],
        out_specs=pl.BlockSpec((blk, d), lambda i: (i, 0)),
        out_shape=jax.ShapeDtypeStruct((n, d), jnp.float32),
    )(va, ea, ca, w, b.reshape(1, d))


def _merge_body(base_ref, agg_ref, w_ref, b_ref, o_ref):
    d = base_ref.shape[1]
    h = (_dotT(base_ref[...], w_ref[:, :d]) +
         _dotT(agg_ref[...], w_ref[:, d:]) + b_ref[...])
    o_ref[...] = base_ref[...] + _ln(jnp.maximum(h, 0.0))


def _merge(base, agg, w, b, blk):
    n, d = base.shape
    return pl.pallas_call(
        _merge_body,
        grid=(n // blk,),
        in_specs=[pl.BlockSpec((blk, d), lambda i: (i, 0)),
                  pl.BlockSpec((blk, d), lambda i: (i, 0)),
                  pl.BlockSpec((d, 2 * d), lambda i: (0, 0)),
                  pl.BlockSpec((1, d), lambda i: (0, 0))],
        out_specs=pl.BlockSpec((blk, d), lambda i: (i, 0)),
        out_shape=jax.ShapeDtypeStruct((n, d), jnp.float32),
    )(base, agg, w, b.reshape(1, d))


def _sc_gather(table, idx):
    n, d = table.shape
    ne = idx.shape[0]
    chunk = 128
    nchunks = ne // chunk
    per_w = nchunks // NW
    extra = nchunks - per_w * NW
    mesh = plsc.VectorSubcoreMesh(core_axis_name="c", subcore_axis_name="s")

    @functools.partial(
        pl.kernel, mesh=mesh,
        out_type=jax.ShapeDtypeStruct((ne, d), jnp.float32),
        scratch_types=[pltpu.VMEM((per_w * chunk,), jnp.int32),
                       pltpu.VMEM((chunk,), jnp.int32),
                       pltpu.VMEM((2, chunk, d), jnp.float32),
                       pltpu.SemaphoreType.DMA,
                       pltpu.SemaphoreType.DMA,
                       pltpu.SemaphoreType.DMA,
                       pltpu.SemaphoreType.DMA],
    )
    def k(table_hbm, idx_hbm, out_hbm, idx_all, idx_x, buf_v,
          g0, g1, w0, w1, ):
        wid = lax.axis_index("s") * NC + lax.axis_index("c")
        gsem, wsem = (g0, g1), (w0, w1)
        pltpu.sync_copy(idx_hbm.at[pl.ds(wid * per_w * chunk, per_w * chunk)],
                        idx_all)

        def start_gather(i, sl):
            return pltpu.async_copy(
                table_hbm.at[idx_all.at[pl.ds(i * chunk, chunk)]],
                buf_v.at[sl], gsem[sl])

        def start_write(i, sl):
            base = (wid * per_w + i) * chunk
            return pltpu.async_copy(buf_v.at[sl],
                                    out_hbm.at[pl.ds(base, chunk), :],
                                    wsem[sl])

        gh = {0: start_gather(0, 0)}
        wh = {}
        for i in range(per_w):
            sl = i % 2
            if i + 1 < per_w:
                if i >= 1:
                    wh[1 - sl].wait()
                gh[(i + 1) % 2] = start_gather(i + 1, (i + 1) % 2)
            gh[sl].wait()
            wh[sl] = start_write(i, sl)
        for sl in (0, 1):
            if sl in wh:
                wh[sl].wait()

        if extra:
            @pl.when(wid < extra)
            def _():
                base = (NW * per_w + wid) * chunk
                pltpu.sync_copy(idx_hbm.at[pl.ds(base, chunk)], idx_x)
                pltpu.async_copy(table_hbm.at[idx_x], buf_v.at[0], g0).wait()
                pltpu.sync_copy(buf_v.at[0], out_hbm.at[pl.ds(base, chunk), :])

    return k(table, idx)


def _sc_segsum(joint, idx, nseg):
    ne, d = joint.shape
    dh = d // NC
    chunk = 128
    nchunks = ne // chunk
    per_s = nchunks // NS
    extra = nchunks - per_s * NS
    rows_per_s = (nseg // NS) // 8 * 8
    tail = nseg - rows_per_s * NS
    zr = 16
    mesh = plsc.VectorSubcoreMesh(core_axis_name="c", subcore_axis_name="s")

    @functools.partial(
        pl.kernel, mesh=mesh,
        out_type=jax.ShapeDtypeStruct((nseg, d), jnp.float32),
        scratch_types=[pltpu.VMEM((chunk,), jnp.int32),
                       pltpu.VMEM((chunk,), jnp.int32),
                       pltpu.VMEM((2, chunk, dh), jnp.float32),
                       pltpu.VMEM((zr, dh), jnp.float32),
                       pltpu.VMEM_SHARED((nseg, dh), jnp.float32),
                       pltpu.SemaphoreType.DMA,
                       pltpu.SemaphoreType.DMA,
                       pltpu.SemaphoreType.DMA,
                       pltpu.SemaphoreType.DMA,
                       pltpu.SemaphoreType.DMA,
                       pltpu.SemaphoreType.DMA],
    )
    def k(joint_hbm, idx_hbm, out_hbm, idx_a, idx_b, buf_v, zbuf, acc_sh,
          i0, i1, r0, r1, s0, s1):
        c = lax.axis_index("c")
        s = lax.axis_index("s")
        idxr = (idx_a, idx_b)
        isem, rsem, ssem = (i0, i1), (r0, r1), (s0, s1)
        row_base = s * rows_per_s
        nz = (rows_per_s + jnp.where(s == NS - 1, tail, 0)) // zr

        def zrow(r, carry):
            def zcol(q, carry2):
                zbuf[r, pl.ds(q * 16, 16)] = jnp.zeros((16,), jnp.float32)
                return carry2
            return lax.fori_loop(0, dh // 16, zcol, carry)
        lax.fori_loop(0, zr, zrow, 0)

        def zdma(t, carry):
            pltpu.sync_copy(zbuf, acc_sh.at[pl.ds(row_base + t * zr, zr)])
            return carry
        lax.fori_loop(0, nz, zdma, 0)
        plsc.subcore_barrier()

        def start_load(i, sl):
            base = (s * per_s + i) * chunk
            ih = pltpu.async_copy(idx_hbm.at[pl.ds(base, chunk)],
                                  idxr[sl], isem[sl])
            rh = pltpu.async_copy(
                joint_hbm.at[pl.ds(base, chunk), pl.ds(c * dh, dh)],
                buf_v.at[sl], rsem[sl])
            return ih, rh

        def start_scatter(sl):
            return pltpu.async_copy(buf_v.at[sl], acc_sh.at[idxr[sl]],
                                    ssem[sl], add=True)

        lh = {0: start_load(0, 0)}
        sh = {}
        for i in range(per_s):
            sl = i % 2
            if i + 1 < per_s:
                if i >= 1:
                    sh[1 - sl].wait()
                lh[(i + 1) % 2] = start_load(i + 1, (i + 1) % 2)
            lh[sl][0].wait()
            lh[sl][1].wait()
            sh[sl] = start_scatter(sl)
        for sl in (0, 1):
            if sl in sh:
                sh[sl].wait()

        if extra:
            @pl.when(s < extra)
            def _():
                base = (NS * per_s + s) * chunk
                pltpu.sync_copy(idx_hbm.at[pl.ds(base, chunk)], idx_a)
                pltpu.sync_copy(
                    joint_hbm.at[pl.ds(base, chunk), pl.ds(c * dh, dh)],
                    buf_v.at[0])
                pltpu.sync_copy(buf_v.at[0], acc_sh.at[idx_a], add=True)
        plsc.subcore_barrier()

        pltpu.sync_copy(
            acc_sh.at[pl.ds(row_base, rows_per_s)],
            out_hbm.at[pl.ds(row_base, rows_per_s), pl.ds(c * dh, dh)])

        if tail:
            @pl.when(s == NS - 1)
            def _():
                tb = NS * rows_per_s
                pltpu.sync_copy(
                    acc_sh.at[pl.ds(tb, tail)],
                    out_hbm.at[pl.ds(tb, tail), pl.ds(c * dh, dh)])

    return k(joint, idx)


def kernel(variable_emb, edge_emb, constraint_emb, W_left, b_left, W_edge,
           W_right, W_join, b_join, W_merge, b_merge, e_u, e_v):
    nu, d = variable_emb.shape
    nv = constraint_emb.shape[0]
    zb = jnp.zeros((d,), jnp.float32)

    var_t = _transform(variable_emb, W_left, b_left, 2000)
    edge_t = _transform(edge_emb, W_edge, zb, 1000)
    con_t = _transform(constraint_emb, W_right, zb, 2000)

    var_agg = _sc_gather(var_t, e_u)

    con_agg = _sc_gather(con_t, e_v)
    joint1 = _joint(var_agg, edge_t, con_agg, W_join, b_join, 1000)
    agg1 = _sc_segsum(joint1, e_v, nv)
    con2 = _merge(con_t, agg1, W_merge, b_merge, 2000)

    con_t2 = _transform(con2, W_right, zb, 2000)
    con_agg2 = _sc_gather(con_t2, e_v)
    joint2 = _joint(var_agg, edge_t, con_agg2, W_join, b_join, 1000)
    agg2 = _sc_segsum(joint2, e_u, nu)
    var2 = _merge(var_t, agg2, W_merge, b_merge, 2000)

    return (var2, con2)
